# Initial kernel scaffold; baseline (speedup 1.0000x reference)
import jax
import jax.numpy as jnp
from jax.experimental import pallas as pl


def kernel(heat, edge_index, W1, b1, a_conv1, gamma0, beta0, a_act0, W2, b2, a_conv2, gamma1, beta1, a_act1):
    raise NotImplementedError("write your pallas kernel here")


# speedup vs baseline: 8.3006x; 8.3006x over previous
import functools

import jax
import jax.numpy as jnp
from jax import lax
from jax.experimental import pallas as pl
from jax.experimental.pallas import tpu as pltpu
from jax.experimental.pallas import tpu_sc as plsc

NCORES = 2
NSUB = 16
CH = 128


def _sc_mesh():
    return plsc.VectorSubcoreMesh(core_axis_name="c", subcore_axis_name="s")


def _sc_degrees(src_r, dst_r, Np, NCH):
    stripe = Np // NSUB
    half = NCH // 2

    @functools.partial(
        pl.kernel,
        out_type=jax.ShapeDtypeStruct((NCORES, 2, Np), jnp.float32),
        mesh=_sc_mesh(),
        scratch_types=[
            pltpu.VMEM((half, CH), jnp.int32),
            pltpu.VMEM((half, CH), jnp.int32),
            pltpu.VMEM((CH,), jnp.float32),
            pltpu.VMEM((stripe,), jnp.float32),
            pltpu.VMEM_SHARED((Np,), jnp.float32),
            pltpu.VMEM_SHARED((Np,), jnp.float32),
        ],
    )
    def deg(src_hbm, dst_hbm, out_hbm, src_v, dst_v, ones_v, zer, acc_s, acc_d):
        c = lax.axis_index("c")
        s = lax.axis_index("s")

        def fill_ones(i, carry):
            ones_v[pl.ds(i * 16, 16)] = jnp.ones((16,), jnp.float32)
            return carry

        lax.fori_loop(0, CH // 16, fill_ones, 0)

        def fill_zeros(i, carry):
            zer[pl.ds(i * 16, 16)] = jnp.zeros((16,), jnp.float32)
            return carry

        lax.fori_loop(0, stripe // 16, fill_zeros, 0)

        base = s * stripe
        pltpu.sync_copy(zer, acc_s.at[pl.ds(base, stripe)])
        pltpu.sync_copy(zer, acc_d.at[pl.ds(base, stripe)])
        pltpu.sync_copy(src_hbm.at[s, pl.ds(c * half, half)], src_v)
        pltpu.sync_copy(dst_hbm.at[s, pl.ds(c * half, half)], dst_v)
        plsc.subcore_barrier()

        def body(j, carry):
            pltpu.sync_copy(ones_v, acc_s.at[src_v.at[j]], add=True)
            pltpu.sync_copy(ones_v, acc_d.at[dst_v.at[j]], add=True)
            return carry

        lax.fori_loop(0, half, body, 0)
        plsc.subcore_barrier()
        pltpu.sync_copy(acc_s.at[pl.ds(base, stripe)], out_hbm.at[c, 0, pl.ds(base, stripe)])
        pltpu.sync_copy(acc_d.at[pl.ds(base, stripe)], out_hbm.at[c, 1, pl.ds(base, stripe)])

    return deg(src_r, dst_r)


def _sc_segsum(h2, src_r, dst_r, Np, Dh, NCH):
    stripe = Np // NSUB
    zch = stripe // CH

    @functools.partial(
        pl.kernel,
        out_type=jax.ShapeDtypeStruct((NCORES, Np, Dh), jnp.float32),
        mesh=_sc_mesh(),
        compiler_params=pltpu.CompilerParams(use_tc_tiling_on_sc=False),
        scratch_types=[
            pltpu.VMEM((NCH, CH), jnp.int32),
            pltpu.VMEM((NCH, CH), jnp.int32),
            pltpu.VMEM((CH, Dh), jnp.float32),
            pltpu.VMEM((CH, Dh), jnp.float32),
            pltpu.VMEM((CH, Dh), jnp.float32),
            pltpu.VMEM_SHARED((Np, Dh), jnp.float32),
            pltpu.SemaphoreType.DMA,
            pltpu.SemaphoreType.DMA,
        ],
    )
    def seg(h_hbm, src_hbm, dst_hbm, out_hbm, src_v, dst_v, buf0, buf1, zer, acc, sem0, sem1):
        c = lax.axis_index("c")
        s = lax.axis_index("s")

        def zrow(i, carry):
            def zcol(j, carry2):
                zer[i, pl.ds(j * 16, 16)] = jnp.zeros((16,), jnp.float32)
                return carry2

            return lax.fori_loop(0, Dh // 16, zcol, carry)

        lax.fori_loop(0, CH, zrow, 0)

        base = s * stripe

        def zcp(i, carry):
            pltpu.sync_copy(zer, acc.at[pl.ds(base + i * CH, CH)])
            return carry

        lax.fori_loop(0, zch, zcp, 0)
        pltpu.sync_copy(src_hbm.at[s], src_v)
        pltpu.sync_copy(dst_hbm.at[s], dst_v)
        plsc.subcore_barrier()

        hc = h_hbm.at[c]

        pltpu.async_copy(hc.at[src_v.at[0]], buf0, sem0)

        def body(g, carry):
            j0 = 2 * g
            pltpu.async_copy(hc.at[src_v.at[j0 + 1]], buf1, sem1)
            pltpu.make_async_copy(hc.at[src_v.at[j0]], buf0, sem0).wait()
            pltpu.sync_copy(buf0, acc.at[dst_v.at[j0]], add=True)

            @pl.when(j0 + 2 < NCH)
            def _():
                pltpu.async_copy(hc.at[src_v.at[j0 + 2]], buf0, sem0)

            pltpu.make_async_copy(hc.at[src_v.at[j0 + 1]], buf1, sem1).wait()
            pltpu.sync_copy(buf1, acc.at[dst_v.at[j0 + 1]], add=True)
            return carry

        lax.fori_loop(0, NCH // 2, body, 0)
        plsc.subcore_barrier()
        pltpu.sync_copy(acc.at[pl.ds(base, stripe)], out_hbm.at[c, pl.ds(base, stripe)])

    return seg(h2, src_r, dst_r)


def _tc_prep(heat_pad, degc, N, Np, D, Dh):
    def body(heat_ref, degc_ref, h0_ref, snd_ref):
        degs = degc_ref[:, 0:1] + degc_ref[:, 2:3]
        degd = degc_ref[:, 1:2] + degc_ref[:, 3:4]
        sn = lax.rsqrt(jnp.maximum(degs, 1.0))
        dn = lax.rsqrt(jnp.maximum(degd, 1.0))
        snd_ref[...] = jnp.concatenate([sn, dn], axis=1)
        hs = heat_ref[...] * sn
        h0_ref[0] = hs[:, :Dh]
        h0_ref[1] = hs[:, Dh:]

    return pl.pallas_call(
        body,
        out_shape=[
            jax.ShapeDtypeStruct((NCORES, Np, Dh), jnp.float32),
            jax.ShapeDtypeStruct((Np, 2), jnp.float32),
        ],
    )(heat_pad, degc)


def _tc_dense(aggp, snd, W, b2, ac2, g2, bt2, aa2, gh_in, N, Np, D, Dh, split_out):
    def body(aggp_ref, snd_ref, w_ref, b_ref, ac_ref, g_ref, bt_ref, aa_ref,
             ghin_ref, h_ref, gh_ref):
        agg = jnp.concatenate([aggp_ref[0], aggp_ref[1]], axis=1)
        dn = snd_ref[:, 1:2]
        y = jnp.dot(agg * dn, w_ref[...], preferred_element_type=jnp.float32)
        y = y + b_ref[...]
        ac = ac_ref[0, 0]
        y = jnp.where(y >= 0.0, y, ac * y)
        rows = lax.broadcasted_iota(jnp.int32, (Np, 1), 0)
        mask = (rows < N).astype(jnp.float32)
        ym = y * mask
        inv_n = 1.0 / N
        mu = jnp.sum(ym, axis=0, keepdims=True) * inv_n
        var = jnp.sum(ym * ym, axis=0, keepdims=True) * inv_n - mu * mu
        z = (y - mu) * lax.rsqrt(var + 1e-5) * g_ref[...] + bt_ref[...]
        aa = aa_ref[0, 0]
        h = jnp.where(z >= 0.0, z, aa * z)
        hm = h * mask
        gh_ref[...] = ghin_ref[...] + jnp.sum(hm, axis=0, keepdims=True)
        if split_out:
            hs = hm * snd_ref[:, 0:1]
            h_ref[0] = hs[:, :Dh]
            h_ref[1] = hs[:, Dh:]
        else:
            h_ref[...] = hm

    h_shape = ((NCORES, Np, Dh) if split_out else (Np, D))
    return pl.pallas_call(
        body,
        out_shape=[
            jax.ShapeDtypeStruct(h_shape, jnp.float32),
            jax.ShapeDtypeStruct((1, D), jnp.float32),
        ],
    )(aggp, snd, W, b2, ac2, g2, bt2, aa2, gh_in)


def kernel(heat, edge_index, W1, b1, a_conv1, gamma0, beta0, a_act0,
           W2, b2, a_conv2, gamma1, beta1, a_act1):
    N, D = heat.shape
    Dh = D // 2
    E = edge_index.shape[1]
    Np = (N // 2048 + 1) * 2048
    NCH = -(-E // (NSUB * CH))
    NCH = -(-NCH // 16) * 16
    Ep = NSUB * NCH * CH
    pad_rows = Np - N

    pad_idx = N + (jnp.arange(Ep - E, dtype=jnp.int32) % pad_rows)
    src_r = jnp.concatenate([edge_index[0], pad_idx]).reshape(NSUB, NCH, CH)
    dst_r = jnp.concatenate([edge_index[1], pad_idx]).reshape(NSUB, NCH, CH)
    heat_pad = jnp.concatenate([heat, jnp.zeros((pad_rows, D), jnp.float32)])

    degp = _sc_degrees(src_r, dst_r, Np, NCH)
    degc = degp.transpose(2, 0, 1).reshape(Np, 2 * NCORES)
    h0, snd = _tc_prep(heat_pad, degc, N, Np, D, Dh)

    b1r, g0r, bt0r = b1.reshape(1, D), gamma0.reshape(1, D), beta0.reshape(1, D)
    b2r, g1r, bt1r = b2.reshape(1, D), gamma1.reshape(1, D), beta1.reshape(1, D)
    ac1, aa0 = a_conv1.reshape(1, 1), a_act0.reshape(1, 1)
    ac2, aa1 = a_conv2.reshape(1, 1), a_act1.reshape(1, 1)

    aggp1 = _sc_segsum(h0, src_r, dst_r, Np, Dh, NCH)
    gh0 = jnp.zeros((1, D), jnp.float32)
    h1s, gh1 = _tc_dense(aggp1, snd, W1, b1r, ac1, g0r, bt0r, aa0, gh0,
                         N, Np, D, Dh, split_out=True)

    aggp2 = _sc_segsum(h1s, src_r, dst_r, Np, Dh, NCH)
    h2, gh = _tc_dense(aggp2, snd, W2, b2r, ac2, g1r, bt1r, aa1, gh1,
                       N, Np, D, Dh, split_out=False)

    return h2[:N], gh


# speedup vs baseline: 8.9184x; 1.0744x over previous
import functools

import jax
import jax.numpy as jnp
from jax import lax
from jax.experimental import pallas as pl
from jax.experimental.pallas import tpu as pltpu
from jax.experimental.pallas import tpu_sc as plsc

NCORES = 2
NSUB = 16
CH = 128


def _sc_mesh():
    return plsc.VectorSubcoreMesh(core_axis_name="c", subcore_axis_name="s")


def _sc_degrees(src_r, dst_r, Np, NCH):
    stripe = Np // NSUB
    half = NCH // 2

    @functools.partial(
        pl.kernel,
        out_type=jax.ShapeDtypeStruct((NCORES, 2, Np), jnp.float32),
        mesh=_sc_mesh(),
        scratch_types=[
            pltpu.VMEM((half, CH), jnp.int32),
            pltpu.VMEM((half, CH), jnp.int32),
            pltpu.VMEM((CH,), jnp.float32),
            pltpu.VMEM((stripe,), jnp.float32),
            pltpu.VMEM_SHARED((Np,), jnp.float32),
            pltpu.VMEM_SHARED((Np,), jnp.float32),
        ],
    )
    def deg(src_hbm, dst_hbm, out_hbm, src_v, dst_v, ones_v, zer, acc_s, acc_d):
        c = lax.axis_index("c")
        s = lax.axis_index("s")

        def fill_ones(i, carry):
            ones_v[pl.ds(i * 16, 16)] = jnp.ones((16,), jnp.float32)
            return carry

        lax.fori_loop(0, CH // 16, fill_ones, 0)

        def fill_zeros(i, carry):
            zer[pl.ds(i * 16, 16)] = jnp.zeros((16,), jnp.float32)
            return carry

        lax.fori_loop(0, stripe // 16, fill_zeros, 0)

        base = s * stripe
        pltpu.sync_copy(zer, acc_s.at[pl.ds(base, stripe)])
        pltpu.sync_copy(zer, acc_d.at[pl.ds(base, stripe)])
        pltpu.sync_copy(src_hbm.at[s, pl.ds(c * half, half)], src_v)
        pltpu.sync_copy(dst_hbm.at[s, pl.ds(c * half, half)], dst_v)
        plsc.subcore_barrier()

        def body(j, carry):
            pltpu.sync_copy(ones_v, acc_s.at[src_v.at[j]], add=True)
            pltpu.sync_copy(ones_v, acc_d.at[dst_v.at[j]], add=True)
            return carry

        lax.fori_loop(0, half, body, 0)
        plsc.subcore_barrier()
        pltpu.sync_copy(acc_s.at[pl.ds(base, stripe)], out_hbm.at[c, 0, pl.ds(base, stripe)])
        pltpu.sync_copy(acc_d.at[pl.ds(base, stripe)], out_hbm.at[c, 1, pl.ds(base, stripe)])

    return deg(src_r, dst_r)


def _sc_segsum(h2, src_r, dst_r, Np, Dh, NCH):
    stripe = Np // NSUB
    zch = stripe // CH

    @functools.partial(
        pl.kernel,
        out_type=jax.ShapeDtypeStruct((NCORES, Np, Dh), jnp.float32),
        mesh=_sc_mesh(),
        compiler_params=pltpu.CompilerParams(use_tc_tiling_on_sc=False),
        scratch_types=[
            pltpu.VMEM((NCH, CH), jnp.int32),
            pltpu.VMEM((NCH, CH), jnp.int32),
            pltpu.VMEM((CH, Dh), jnp.float32),
            pltpu.VMEM((CH, Dh), jnp.float32),
            pltpu.VMEM((CH, Dh), jnp.float32),
            pltpu.VMEM((CH, Dh), jnp.float32),
            pltpu.VMEM((CH, Dh), jnp.float32),
            pltpu.VMEM_SHARED((Np, Dh), jnp.float32),
            pltpu.SemaphoreType.DMA,
            pltpu.SemaphoreType.DMA,
            pltpu.SemaphoreType.DMA,
            pltpu.SemaphoreType.DMA,
            pltpu.SemaphoreType.DMA,
            pltpu.SemaphoreType.DMA,
            pltpu.SemaphoreType.DMA,
            pltpu.SemaphoreType.DMA,
        ],
    )
    def seg(h_hbm, src_hbm, dst_hbm, out_hbm, src_v, dst_v,
            buf0, buf1, buf2, buf3, zer, acc,
            gs0, gs1, gs2, gs3, ss0, ss1, ss2, ss3):
        c = lax.axis_index("c")
        s = lax.axis_index("s")

        def zrow(i, carry):
            def zcol(j, carry2):
                zer[i, pl.ds(j * 16, 16)] = jnp.zeros((16,), jnp.float32)
                return carry2

            return lax.fori_loop(0, Dh // 16, zcol, carry)

        lax.fori_loop(0, CH, zrow, 0)

        base = s * stripe

        def zcp(i, carry):
            pltpu.sync_copy(zer, acc.at[pl.ds(base + i * CH, CH)])
            return carry

        lax.fori_loop(0, zch, zcp, 0)
        pltpu.sync_copy(src_hbm.at[s], src_v)
        pltpu.sync_copy(dst_hbm.at[s], dst_v)
        plsc.subcore_barrier()

        hc = h_hbm.at[c]
        bufs = (buf0, buf1, buf2, buf3)
        gsems = (gs0, gs1, gs2, gs3)
        ssems = (ss0, ss1, ss2, ss3)

        for b in range(4):
            pltpu.async_copy(hc.at[src_v.at[b]], bufs[b], gsems[b])

        def body(g, carry):
            j0 = 4 * g
            for b in range(4):
                j = j0 + b
                pltpu.make_async_copy(hc.at[src_v.at[j]], bufs[b], gsems[b]).wait()
                pltpu.async_copy(bufs[b], acc.at[dst_v.at[j]], ssems[b], add=True)
            for b in range(4):
                j = j0 + b

                @pl.when(j + 4 < NCH)
                def _(j=j, b=b):
                    pltpu.make_async_copy(bufs[b], acc.at[dst_v.at[j]], ssems[b]).wait()
                    pltpu.async_copy(hc.at[src_v.at[j + 4]], bufs[b], gsems[b])

            return carry

        lax.fori_loop(0, NCH // 4, body, 0)
        for b in range(4):
            pltpu.make_async_copy(bufs[b], acc.at[dst_v.at[NCH - 4 + b]], ssems[b]).wait()
        plsc.subcore_barrier()
        pltpu.sync_copy(acc.at[pl.ds(base, stripe)], out_hbm.at[c, pl.ds(base, stripe)])

    return seg(h2, src_r, dst_r)


def _tc_prep(heat_pad, degc, N, Np, D, Dh):
    def body(heat_ref, degc_ref, h0_ref, snd_ref):
        degs = degc_ref[:, 0:1] + degc_ref[:, 2:3]
        degd = degc_ref[:, 1:2] + degc_ref[:, 3:4]
        sn = lax.rsqrt(jnp.maximum(degs, 1.0))
        dn = lax.rsqrt(jnp.maximum(degd, 1.0))
        snd_ref[...] = jnp.concatenate([sn, dn], axis=1)
        hs = heat_ref[...] * sn
        h0_ref[0] = hs[:, :Dh]
        h0_ref[1] = hs[:, Dh:]

    return pl.pallas_call(
        body,
        out_shape=[
            jax.ShapeDtypeStruct((NCORES, Np, Dh), jnp.float32),
            jax.ShapeDtypeStruct((Np, 2), jnp.float32),
        ],
    )(heat_pad, degc)


def _tc_dense(aggp, snd, W, b2, ac2, g2, bt2, aa2, gh_in, N, Np, D, Dh, split_out):
    def body(aggp_ref, snd_ref, w_ref, b_ref, ac_ref, g_ref, bt_ref, aa_ref,
             ghin_ref, h_ref, gh_ref):
        agg = jnp.concatenate([aggp_ref[0], aggp_ref[1]], axis=1)
        dn = snd_ref[:, 1:2]
        y = jnp.dot(agg * dn, w_ref[...], preferred_element_type=jnp.float32)
        y = y + b_ref[...]
        ac = ac_ref[0, 0]
        y = jnp.where(y >= 0.0, y, ac * y)
        rows = lax.broadcasted_iota(jnp.int32, (Np, 1), 0)
        mask = (rows < N).astype(jnp.float32)
        ym = y * mask
        inv_n = 1.0 / N
        mu = jnp.sum(ym, axis=0, keepdims=True) * inv_n
        var = jnp.sum(ym * ym, axis=0, keepdims=True) * inv_n - mu * mu
        z = (y - mu) * lax.rsqrt(var + 1e-5) * g_ref[...] + bt_ref[...]
        aa = aa_ref[0, 0]
        h = jnp.where(z >= 0.0, z, aa * z)
        hm = h * mask
        gh_ref[...] = ghin_ref[...] + jnp.sum(hm, axis=0, keepdims=True)
        if split_out:
            hs = hm * snd_ref[:, 0:1]
            h_ref[0] = hs[:, :Dh]
            h_ref[1] = hs[:, Dh:]
        else:
            h_ref[...] = hm

    h_shape = ((NCORES, Np, Dh) if split_out else (Np, D))
    return pl.pallas_call(
        body,
        out_shape=[
            jax.ShapeDtypeStruct(h_shape, jnp.float32),
            jax.ShapeDtypeStruct((1, D), jnp.float32),
        ],
    )(aggp, snd, W, b2, ac2, g2, bt2, aa2, gh_in)


def kernel(heat, edge_index, W1, b1, a_conv1, gamma0, beta0, a_act0,
           W2, b2, a_conv2, gamma1, beta1, a_act1):
    N, D = heat.shape
    Dh = D // 2
    E = edge_index.shape[1]
    Np = (N // 2048 + 1) * 2048
    NCH = -(-E // (NSUB * CH))
    NCH = -(-NCH // 16) * 16
    Ep = NSUB * NCH * CH
    pad_rows = Np - N

    pad_idx = N + (jnp.arange(Ep - E, dtype=jnp.int32) % pad_rows)
    src_r = jnp.concatenate([edge_index[0], pad_idx]).reshape(NSUB, NCH, CH)
    dst_r = jnp.concatenate([edge_index[1], pad_idx]).reshape(NSUB, NCH, CH)
    heat_pad = jnp.concatenate([heat, jnp.zeros((pad_rows, D), jnp.float32)])

    degp = _sc_degrees(src_r, dst_r, Np, NCH)
    degc = degp.transpose(2, 0, 1).reshape(Np, 2 * NCORES)
    h0, snd = _tc_prep(heat_pad, degc, N, Np, D, Dh)

    b1r, g0r, bt0r = b1.reshape(1, D), gamma0.reshape(1, D), beta0.reshape(1, D)
    b2r, g1r, bt1r = b2.reshape(1, D), gamma1.reshape(1, D), beta1.reshape(1, D)
    ac1, aa0 = a_conv1.reshape(1, 1), a_act0.reshape(1, 1)
    ac2, aa1 = a_conv2.reshape(1, 1), a_act1.reshape(1, 1)

    aggp1 = _sc_segsum(h0, src_r, dst_r, Np, Dh, NCH)
    gh0 = jnp.zeros((1, D), jnp.float32)
    h1s, gh1 = _tc_dense(aggp1, snd, W1, b1r, ac1, g0r, bt0r, aa0, gh0,
                         N, Np, D, Dh, split_out=True)

    aggp2 = _sc_segsum(h1s, src_r, dst_r, Np, Dh, NCH)
    h2, gh = _tc_dense(aggp2, snd, W2, b2r, ac2, g1r, bt1r, aa1, gh1,
                       N, Np, D, Dh, split_out=False)

    return h2[:N], gh


# speedup vs baseline: 9.5245x; 1.0680x over previous
import functools

import jax
import jax.numpy as jnp
from jax import lax
from jax.experimental import pallas as pl
from jax.experimental.pallas import tpu as pltpu
from jax.experimental.pallas import tpu_sc as plsc

NCORES = 2
NSUB = 16
CH = 128


def _sc_mesh():
    return plsc.VectorSubcoreMesh(core_axis_name="c", subcore_axis_name="s")


def _sc_degrees(src_r, dst_r, Np, NCH):
    stripe = Np // NSUB
    half = NCH // 2

    @functools.partial(
        pl.kernel,
        out_type=jax.ShapeDtypeStruct((NCORES, 2, Np), jnp.float32),
        mesh=_sc_mesh(),
        scratch_types=[
            pltpu.VMEM((half, CH), jnp.int32),
            pltpu.VMEM((half, CH), jnp.int32),
            pltpu.VMEM((CH,), jnp.float32),
            pltpu.VMEM((stripe,), jnp.float32),
            pltpu.VMEM_SHARED((Np,), jnp.float32),
            pltpu.VMEM_SHARED((Np,), jnp.float32),
        ],
    )
    def deg(src_hbm, dst_hbm, out_hbm, src_v, dst_v, ones_v, zer, acc_s, acc_d):
        c = lax.axis_index("c")
        s = lax.axis_index("s")

        def fill_ones(i, carry):
            ones_v[pl.ds(i * 16, 16)] = jnp.ones((16,), jnp.float32)
            return carry

        lax.fori_loop(0, CH // 16, fill_ones, 0)

        def fill_zeros(i, carry):
            zer[pl.ds(i * 16, 16)] = jnp.zeros((16,), jnp.float32)
            return carry

        lax.fori_loop(0, stripe // 16, fill_zeros, 0)

        base = s * stripe
        pltpu.sync_copy(zer, acc_s.at[pl.ds(base, stripe)])
        pltpu.sync_copy(zer, acc_d.at[pl.ds(base, stripe)])
        pltpu.sync_copy(src_hbm.at[s, pl.ds(c * half, half)], src_v)
        pltpu.sync_copy(dst_hbm.at[s, pl.ds(c * half, half)], dst_v)
        plsc.subcore_barrier()

        def body(j, carry):
            pltpu.sync_copy(ones_v, acc_s.at[src_v.at[j]], add=True)
            pltpu.sync_copy(ones_v, acc_d.at[dst_v.at[j]], add=True)
            return carry

        lax.fori_loop(0, half, body, 0)
        plsc.subcore_barrier()
        pltpu.sync_copy(acc_s.at[pl.ds(base, stripe)], out_hbm.at[c, 0, pl.ds(base, stripe)])
        pltpu.sync_copy(acc_d.at[pl.ds(base, stripe)], out_hbm.at[c, 1, pl.ds(base, stripe)])

    return deg(src_r, dst_r)


def _sc_segsum(h2, src_r, dst_r, Np, Dh, NCH):
    stripe = Np // NSUB
    zch = stripe // CH

    @functools.partial(
        pl.kernel,
        out_type=jax.ShapeDtypeStruct((NCORES, Np, Dh), jnp.float32),
        mesh=_sc_mesh(),
        compiler_params=pltpu.CompilerParams(use_tc_tiling_on_sc=False),
        scratch_types=[
            pltpu.VMEM((NCH, CH), jnp.int32),
            pltpu.VMEM((NCH, CH), jnp.int32),
            pltpu.VMEM((CH, Dh), jnp.float32),
            pltpu.VMEM((CH, Dh), jnp.float32),
            pltpu.VMEM((CH, Dh), jnp.float32),
            pltpu.VMEM((CH, Dh), jnp.float32),
            pltpu.VMEM((CH, Dh), jnp.float32),
            pltpu.VMEM_SHARED((Np, Dh), jnp.float32),
            pltpu.SemaphoreType.DMA,
            pltpu.SemaphoreType.DMA,
            pltpu.SemaphoreType.DMA,
            pltpu.SemaphoreType.DMA,
            pltpu.SemaphoreType.DMA,
            pltpu.SemaphoreType.DMA,
            pltpu.SemaphoreType.DMA,
            pltpu.SemaphoreType.DMA,
        ],
    )
    def seg(h_hbm, src_hbm, dst_hbm, out_hbm, src_v, dst_v,
            buf0, buf1, buf2, buf3, zer, acc,
            gs0, gs1, gs2, gs3, ss0, ss1, ss2, ss3):
        c = lax.axis_index("c")
        s = lax.axis_index("s")

        def zrow(i, carry):
            def zcol(j, carry2):
                zer[i, pl.ds(j * 16, 16)] = jnp.zeros((16,), jnp.float32)
                return carry2

            return lax.fori_loop(0, Dh // 16, zcol, carry)

        lax.fori_loop(0, CH, zrow, 0)

        base = s * stripe

        def zcp(i, carry):
            pltpu.sync_copy(zer, acc.at[pl.ds(base + i * CH, CH)])
            return carry

        lax.fori_loop(0, zch, zcp, 0)
        pltpu.sync_copy(src_hbm.at[s], src_v)
        pltpu.sync_copy(dst_hbm.at[s], dst_v)
        plsc.subcore_barrier()

        hc = h_hbm.at[c]
        bufs = (buf0, buf1, buf2, buf3)
        gsems = (gs0, gs1, gs2, gs3)
        ssems = (ss0, ss1, ss2, ss3)

        for b in range(4):
            pltpu.async_copy(hc.at[src_v.at[b]], bufs[b], gsems[b])

        def body(g, carry):
            j0 = 4 * g
            for b in range(4):
                j = j0 + b
                pltpu.make_async_copy(hc.at[src_v.at[j]], bufs[b], gsems[b]).wait()
                pltpu.async_copy(bufs[b], acc.at[dst_v.at[j]], ssems[b], add=True)
            for b in range(4):
                j = j0 + b

                @pl.when(j + 4 < NCH)
                def _(j=j, b=b):
                    pltpu.make_async_copy(bufs[b], acc.at[dst_v.at[j]], ssems[b]).wait()
                    pltpu.async_copy(hc.at[src_v.at[j + 4]], bufs[b], gsems[b])

            return carry

        lax.fori_loop(0, NCH // 4, body, 0)
        for b in range(4):
            pltpu.make_async_copy(bufs[b], acc.at[dst_v.at[NCH - 4 + b]], ssems[b]).wait()
        plsc.subcore_barrier()
        pltpu.sync_copy(acc.at[pl.ds(base, stripe)], out_hbm.at[c, pl.ds(base, stripe)])

    return seg(h2, src_r, dst_r)


def _pack_split(hs, out_ref, Np, Dh):
    hr = hs.reshape(Np // 2, 2, 2 * Dh)
    ev = hr[:, 0, :]
    od = hr[:, 1, :]
    out_ref[0] = jnp.concatenate([ev[:, :Dh], od[:, :Dh]], axis=1)
    out_ref[1] = jnp.concatenate([ev[:, Dh:], od[:, Dh:]], axis=1)


def _unpack_split(a, Np, Dh):
    ev = jnp.concatenate([a[0, :, :Dh], a[1, :, :Dh]], axis=1)
    od = jnp.concatenate([a[0, :, Dh:], a[1, :, Dh:]], axis=1)
    return jnp.concatenate([ev[:, None, :], od[:, None, :]], axis=1).reshape(Np, 2 * Dh)


def _tc_prep(heat_pad, degc, N, Np, D, Dh):
    def body(heat_ref, degc_ref, h0_ref, snd_ref):
        degs = degc_ref[:, 0:1] + degc_ref[:, 2:3]
        degd = degc_ref[:, 1:2] + degc_ref[:, 3:4]
        sn = lax.rsqrt(jnp.maximum(degs, 1.0))
        dn = lax.rsqrt(jnp.maximum(degd, 1.0))
        snd_ref[...] = jnp.concatenate([sn, dn], axis=1)
        hs = heat_ref[...] * sn
        _pack_split(hs, h0_ref, Np, Dh)

    return pl.pallas_call(
        body,
        out_shape=[
            jax.ShapeDtypeStruct((NCORES, Np // 2, D), jnp.float32),
            jax.ShapeDtypeStruct((Np, 2), jnp.float32),
        ],
    )(heat_pad, degc)


def _tc_dense(aggp, snd, W, b2, ac2, g2, bt2, aa2, gh_in, N, Np, D, Dh, split_out):
    def body(aggp_ref, snd_ref, w_ref, b_ref, ac_ref, g_ref, bt_ref, aa_ref,
             ghin_ref, h_ref, gh_ref):
        agg = _unpack_split(aggp_ref[...], Np, Dh)
        dn = snd_ref[:, 1:2]
        y = jnp.dot(agg * dn, w_ref[...], preferred_element_type=jnp.float32)
        y = y + b_ref[...]
        ac = ac_ref[0, 0]
        y = jnp.where(y >= 0.0, y, ac * y)
        rows = lax.broadcasted_iota(jnp.int32, (Np, 1), 0)
        mask = (rows < N).astype(jnp.float32)
        ym = y * mask
        inv_n = 1.0 / N
        mu = jnp.sum(ym, axis=0, keepdims=True) * inv_n
        var = jnp.sum(ym * ym, axis=0, keepdims=True) * inv_n - mu * mu
        z = (y - mu) * lax.rsqrt(var + 1e-5) * g_ref[...] + bt_ref[...]
        aa = aa_ref[0, 0]
        h = jnp.where(z >= 0.0, z, aa * z)
        hm = h * mask
        gh_ref[...] = ghin_ref[...] + jnp.sum(hm, axis=0, keepdims=True)
        if split_out:
            hs = hm * snd_ref[:, 0:1]
            _pack_split(hs, h_ref, Np, Dh)
        else:
            h_ref[...] = hm

    h_shape = ((NCORES, Np // 2, D) if split_out else (Np, D))
    return pl.pallas_call(
        body,
        out_shape=[
            jax.ShapeDtypeStruct(h_shape, jnp.float32),
            jax.ShapeDtypeStruct((1, D), jnp.float32),
        ],
    )(aggp, snd, W, b2, ac2, g2, bt2, aa2, gh_in)


def kernel(heat, edge_index, W1, b1, a_conv1, gamma0, beta0, a_act0,
           W2, b2, a_conv2, gamma1, beta1, a_act1):
    N, D = heat.shape
    Dh = D // 2
    E = edge_index.shape[1]
    Np = (N // 2048 + 1) * 2048
    NCH = -(-E // (NSUB * CH))
    NCH = -(-NCH // 16) * 16
    Ep = NSUB * NCH * CH
    pad_rows = Np - N

    pad_idx = N + (jnp.arange(Ep - E, dtype=jnp.int32) % pad_rows)
    src_r = jnp.concatenate([edge_index[0], pad_idx]).reshape(NSUB, NCH, CH)
    dst_r = jnp.concatenate([edge_index[1], pad_idx]).reshape(NSUB, NCH, CH)
    heat_pad = jnp.concatenate([heat, jnp.zeros((pad_rows, D), jnp.float32)])

    degp = _sc_degrees(src_r, dst_r, Np, NCH)
    degc = degp.transpose(2, 0, 1).reshape(Np, 2 * NCORES)
    h0, snd = _tc_prep(heat_pad, degc, N, Np, D, Dh)

    b1r, g0r, bt0r = b1.reshape(1, D), gamma0.reshape(1, D), beta0.reshape(1, D)
    b2r, g1r, bt1r = b2.reshape(1, D), gamma1.reshape(1, D), beta1.reshape(1, D)
    ac1, aa0 = a_conv1.reshape(1, 1), a_act0.reshape(1, 1)
    ac2, aa1 = a_conv2.reshape(1, 1), a_act1.reshape(1, 1)

    aggp1 = _sc_segsum(h0.reshape(NCORES, Np, Dh), src_r, dst_r, Np, Dh, NCH)
    gh0 = jnp.zeros((1, D), jnp.float32)
    h1s, gh1 = _tc_dense(aggp1.reshape(NCORES, Np // 2, D), snd, W1, b1r, ac1,
                         g0r, bt0r, aa0, gh0, N, Np, D, Dh, split_out=True)

    aggp2 = _sc_segsum(h1s.reshape(NCORES, Np, Dh), src_r, dst_r, Np, Dh, NCH)
    h2, gh = _tc_dense(aggp2.reshape(NCORES, Np // 2, D), snd, W2, b2r, ac2,
                       g1r, bt1r, aa1, gh1, N, Np, D, Dh, split_out=False)

    return h2[:N], gh
